# baseline (speedup 1.0000x reference)
import functools

import jax
import jax.numpy as jnp
from jax import lax
from jax.experimental import pallas as pl
from jax.experimental.pallas import tpu as pltpu
from jax.experimental.pallas import tpu_sc as plsc

D_MODEL = 1024
MAXLEN = 2048
B = 4 * 2048
NC, NS, L = 2, 16, 16
NW = NC * NS
BPW = B // NW
CH = 16
NCHUNK = BPW // CH
NB = 6
LOOKAHEAD = 2

_mesh = plsc.VectorSubcoreMesh(core_axis_name="c", subcore_axis_name="s")


@functools.partial(
    pl.kernel,
    mesh=_mesh,
    out_type=[
        jax.ShapeDtypeStruct((B, D_MODEL), jnp.float32),
        jax.ShapeDtypeStruct((B, D_MODEL), jnp.float32),
    ],
    scratch_types=(
        [pltpu.VMEM((BPW,), jnp.int32),
         pltpu.VMEM((BPW,), jnp.int32)]
        + [pltpu.VMEM((CH, D_MODEL), jnp.float32) for _ in range(NB)]
        + [pltpu.SemaphoreType.DMA for _ in range(2 * NB)]
    ),
)
def _emb_lookup(idx_hbm, pe_k_hbm, pe_v_hbm, out_k_hbm, out_v_hbm,
                idx_v, cl_v, *bufs_sems):
    bufs = bufs_sems[:NB]
    gsems = bufs_sems[NB:2 * NB]
    wsems = bufs_sems[2 * NB:]
    wid = lax.axis_index("s") * NC + lax.axis_index("c")
    base = wid * BPW
    pltpu.sync_copy(idx_hbm.at[pl.ds(base, BPW)], idx_v)
    for i in range(BPW // L):
        v = idx_v[pl.ds(i * L, L)]
        cl_v[pl.ds(i * L, L)] = jnp.clip(v, -MAXLEN, MAXLEN - 1) + MAXLEN

    jobs = ([(pe_k_hbm, out_k_hbm, c) for c in range(NCHUNK)]
            + [(pe_v_hbm, out_v_hbm, c) for c in range(NCHUNK)])
    NJ = len(jobs)

    def gather(j):
        table, _, c = jobs[j]
        s = j % NB
        return pltpu.async_copy(
            table.at[cl_v.at[pl.ds(c * CH, CH)]], bufs[s], gsems[s])

    def writeback(j):
        _, out, c = jobs[j]
        s = j % NB
        return pltpu.async_copy(bufs[s], out.at[pl.ds(base + c * CH, CH)],
                                wsems[s])

    gpend = [None] * NB
    wpend = [None] * NB
    for j in range(NJ + LOOKAHEAD):
        if j < NJ:
            s = j % NB
            if wpend[s] is not None:
                wpend[s].wait()
            gpend[s] = gather(j)
        jj = j - LOOKAHEAD
        if jj >= 0:
            sp = jj % NB
            gpend[sp].wait()
            wpend[sp] = writeback(jj)
    for jj in range(NJ - LOOKAHEAD, NJ):
        wpend[jj % NB].wait()


def kernel(pos_seq, pe_k, pe_v):
    shp = pos_seq.shape
    idx = pos_seq.reshape(-1).astype(jnp.int32)
    out_k, out_v = _emb_lookup(idx, pe_k, pe_v)
    return (out_k.reshape(*shp, D_MODEL), out_v.reshape(*shp, D_MODEL))


# speedup vs baseline: 1.0020x; 1.0020x over previous
import functools

import jax
import jax.numpy as jnp
from jax import lax
from jax.experimental import pallas as pl
from jax.experimental.pallas import tpu as pltpu
from jax.experimental.pallas import tpu_sc as plsc

D_MODEL = 1024
MAXLEN = 2048
B = 4 * 2048
NC, NS, L = 2, 16, 16
NW = NC * NS
BPW = B // NW
CH = 16
NCHUNK = BPW // CH
NB = 6
LOOKAHEAD = 3

_mesh = plsc.VectorSubcoreMesh(core_axis_name="c", subcore_axis_name="s")


@functools.partial(
    pl.kernel,
    mesh=_mesh,
    out_type=[
        jax.ShapeDtypeStruct((B, D_MODEL), jnp.float32),
        jax.ShapeDtypeStruct((B, D_MODEL), jnp.float32),
    ],
    scratch_types=(
        [pltpu.VMEM((BPW,), jnp.int32),
         pltpu.VMEM((BPW,), jnp.int32)]
        + [pltpu.VMEM((CH, D_MODEL), jnp.float32) for _ in range(NB)]
        + [pltpu.SemaphoreType.DMA for _ in range(2 * NB)]
    ),
)
def _emb_lookup(idx_hbm, pe_k_hbm, pe_v_hbm, out_k_hbm, out_v_hbm,
                idx_v, cl_v, *bufs_sems):
    bufs = bufs_sems[:NB]
    gsems = bufs_sems[NB:2 * NB]
    wsems = bufs_sems[2 * NB:]
    wid = lax.axis_index("s") * NC + lax.axis_index("c")
    base = wid * BPW
    pltpu.sync_copy(idx_hbm.at[pl.ds(base, BPW)], idx_v)
    for i in range(BPW // L):
        v = idx_v[pl.ds(i * L, L)]
        cl_v[pl.ds(i * L, L)] = jnp.clip(v, -MAXLEN, MAXLEN - 1) + MAXLEN

    jobs = [t for c in range(NCHUNK)
            for t in ((pe_k_hbm, out_k_hbm, c), (pe_v_hbm, out_v_hbm, c))]
    NJ = len(jobs)

    def gather(j):
        table, _, c = jobs[j]
        s = j % NB
        return pltpu.async_copy(
            table.at[cl_v.at[pl.ds(c * CH, CH)]], bufs[s], gsems[s])

    def writeback(j):
        _, out, c = jobs[j]
        s = j % NB
        return pltpu.async_copy(bufs[s], out.at[pl.ds(base + c * CH, CH)],
                                wsems[s])

    gpend = [None] * NB
    wpend = [None] * NB
    for j in range(NJ + LOOKAHEAD):
        if j < NJ:
            s = j % NB
            if wpend[s] is not None:
                wpend[s].wait()
            gpend[s] = gather(j)
        jj = j - LOOKAHEAD
        if jj >= 0:
            sp = jj % NB
            gpend[sp].wait()
            wpend[sp] = writeback(jj)
    for jj in range(NJ - LOOKAHEAD, NJ):
        wpend[jj % NB].wait()


def kernel(pos_seq, pe_k, pe_v):
    shp = pos_seq.shape
    idx = pos_seq.reshape(-1).astype(jnp.int32)
    out_k, out_v = _emb_lookup(idx, pe_k, pe_v)
    return (out_k.reshape(*shp, D_MODEL), out_v.reshape(*shp, D_MODEL))


# speedup vs baseline: 1.0020x; 1.0000x over previous
import functools

import jax
import jax.numpy as jnp
from jax import lax
from jax.experimental import pallas as pl
from jax.experimental.pallas import tpu as pltpu
from jax.experimental.pallas import tpu_sc as plsc

D_MODEL = 1024
MAXLEN = 2048
B = 4 * 2048
NC, NS, L = 2, 16, 16
NW = NC * NS
BPW = B // NW
CH = 16
NCHUNK = BPW // CH
NB = 6
LOOKAHEAD = 3

_mesh = plsc.VectorSubcoreMesh(core_axis_name="c", subcore_axis_name="s")


@functools.partial(
    pl.kernel,
    mesh=_mesh,
    out_type=[
        jax.ShapeDtypeStruct((B, D_MODEL), jnp.float32),
        jax.ShapeDtypeStruct((B, D_MODEL), jnp.float32),
    ],
    scratch_types=(
        [pltpu.VMEM((BPW,), jnp.int32),
         pltpu.VMEM((BPW,), jnp.int32)]
        + [pltpu.VMEM((CH, D_MODEL), jnp.float32) for _ in range(NB)]
        + [pltpu.SemaphoreType.DMA for _ in range(2 * NB)]
    ),
)
def _emb_lookup(idx_hbm, pe_k_hbm, pe_v_hbm, out_k_hbm, out_v_hbm,
                idx_v, cl_v, *bufs_sems):
    bufs = bufs_sems[:NB]
    gsems = bufs_sems[NB:2 * NB]
    wsems = bufs_sems[2 * NB:]
    wid = lax.axis_index("s") * NC + lax.axis_index("c")
    base = wid * BPW
    pltpu.sync_copy(idx_hbm.at[pl.ds(base, BPW)], idx_v)
    for i in range(BPW // L):
        v = idx_v[pl.ds(i * L, L)]
        cl_v[pl.ds(i * L, L)] = jnp.clip(v, -MAXLEN, MAXLEN - 1) + MAXLEN

    jobs = [t for c in range(NCHUNK)
            for t in ((pe_k_hbm, out_k_hbm, c), (pe_v_hbm, out_v_hbm, c))]
    NJ = len(jobs)

    def gather(j):
        table, _, c = jobs[j]
        s = j % NB
        idx_vec = cl_v[pl.ds(c * CH, CH)]
        return pltpu.async_copy(table.at[idx_vec], bufs[s], gsems[s])

    def writeback(j):
        _, out, c = jobs[j]
        s = j % NB
        return pltpu.async_copy(bufs[s], out.at[pl.ds(base + c * CH, CH)],
                                wsems[s])

    gpend = [None] * NB
    wpend = [None] * NB
    for j in range(NJ + LOOKAHEAD):
        if j < NJ:
            s = j % NB
            if wpend[s] is not None:
                wpend[s].wait()
            gpend[s] = gather(j)
        jj = j - LOOKAHEAD
        if jj >= 0:
            sp = jj % NB
            gpend[sp].wait()
            wpend[sp] = writeback(jj)
    for jj in range(NJ - LOOKAHEAD, NJ):
        wpend[jj % NB].wait()


def kernel(pos_seq, pe_k, pe_v):
    shp = pos_seq.shape
    idx = pos_seq.reshape(-1).astype(jnp.int32)
    out_k, out_v = _emb_lookup(idx, pe_k, pe_v)
    return (out_k.reshape(*shp, D_MODEL), out_v.reshape(*shp, D_MODEL))


# speedup vs baseline: 1.0060x; 1.0040x over previous
import functools

import jax
import jax.numpy as jnp
from jax import lax
from jax.experimental import pallas as pl
from jax.experimental.pallas import tpu as pltpu
from jax.experimental.pallas import tpu_sc as plsc

D_MODEL = 1024
MAXLEN = 2048
B = 4 * 2048
NC, NS, L = 2, 16, 16
NW = NC * NS
BPW = B // NW
CH = 16
NCHUNK = BPW // CH
NB = 6
LOOKAHEAD = 3

_mesh = plsc.VectorSubcoreMesh(core_axis_name="c", subcore_axis_name="s")


@functools.partial(
    pl.kernel,
    mesh=_mesh,
    out_type=[
        jax.ShapeDtypeStruct((B, D_MODEL), jnp.float32),
        jax.ShapeDtypeStruct((B, D_MODEL), jnp.float32),
    ],
    scratch_types=(
        [pltpu.VMEM((BPW,), jnp.int32),
         pltpu.VMEM((BPW,), jnp.int32)]
        + [pltpu.VMEM((CH, D_MODEL), jnp.float32) for _ in range(NB)]
        + [pltpu.SemaphoreType.DMA for _ in range(2 * NB)]
    ),
)
def _emb_lookup(idx_hbm, pe_k_hbm, pe_v_hbm, out_k_hbm, out_v_hbm,
                idx_v, cl_v, *bufs_sems):
    bufs = bufs_sems[:NB]
    gsems = bufs_sems[NB:2 * NB]
    wsems = bufs_sems[2 * NB:]
    wid = lax.axis_index("s") * NC + lax.axis_index("c")
    base = wid * BPW
    pltpu.sync_copy(idx_hbm.at[pl.ds(base, BPW)], idx_v)
    for i in range(BPW // L):
        v = idx_v[pl.ds(i * L, L)]
        cl_v[pl.ds(i * L, L)] = jnp.clip(v, -MAXLEN, MAXLEN - 1) + MAXLEN

    jobs = [t for c in range(NCHUNK)
            for t in ((pe_k_hbm, out_k_hbm, c), (pe_v_hbm, out_v_hbm, c))]
    NJ = len(jobs)

    def gather(j):
        table, _, c = jobs[j]
        s = j % NB
        idx_vec = cl_v[pl.ds(c * CH, CH)]
        return pltpu.async_copy(table.at[idx_vec], bufs[s], gsems[s])

    def writeback(j):
        _, out, c = jobs[j]
        s = j % NB
        return pltpu.async_copy(bufs[s], out.at[pl.ds(base + c * CH, CH)],
                                wsems[s])

    gpend = [None] * NB
    wpend = [None] * NB
    for j in range(NJ + LOOKAHEAD):
        if j < NJ:
            s = j % NB
            if wpend[s] is not None:
                wpend[s].wait()
            gpend[s] = gather(j)
        jj = j - LOOKAHEAD
        if jj >= 0:
            sp = jj % NB
            gpend[sp].wait()
            wpend[sp] = writeback(jj)
    for jj in range(NJ - NB, NJ):
        wpend[jj % NB].wait()


def kernel(pos_seq, pe_k, pe_v):
    shp = pos_seq.shape
    idx = pos_seq.reshape(-1).astype(jnp.int32)
    out_k, out_v = _emb_lookup(idx, pe_k, pe_v)
    return (out_k.reshape(*shp, D_MODEL), out_v.reshape(*shp, D_MODEL))
